# baseline (speedup 1.0000x reference)
import functools

import jax
import jax.numpy as jnp
import numpy as np
from jax import lax
from jax.experimental import pallas as pl
from jax.experimental.pallas import tpu as pltpu
from jax.experimental.pallas import tpu_sc as plsc

_N = 10000
_E = 320000
_NPAD = 10240
_NC = 2
_NS = 16
_K = 128
_CHP = 160
_CHB = 20
_EP = _NS * _CHP * _K
_TRASH = 10200
_RSUB = _NPAD // _NS
_R = 2048
_BN_SCALE = float(1.0 / np.sqrt(1.0 + 1e-5))

_mesh = plsc.VectorSubcoreMesh(
    core_axis_name="c", subcore_axis_name="s", num_cores=_NC, num_subcores=_NS)


def _make_sc_segsum(W, with_counts=False):
    out_type = [jax.ShapeDtypeStruct((_NPAD, W), jnp.float32),
                jax.ShapeDtypeStruct((_NPAD, W), jnp.float32)]
    scratch = [pltpu.VMEM((_CHB, _K), jnp.int32),
               pltpu.VMEM((_CHB, _K), jnp.int32),
               pltpu.VMEM((_K, W), jnp.float32),
               pltpu.VMEM((_K, W), jnp.float32),
               pltpu.VMEM_SHARED((_NPAD, W), jnp.float32),
               pltpu.VMEM_SHARED((_NPAD, W), jnp.float32),
               pltpu.SemaphoreType.DMA,
               pltpu.SemaphoreType.DMA,
               pltpu.SemaphoreType.DMA]
    if with_counts:
        out_type += [jax.ShapeDtypeStruct((_NPAD, 16), jnp.float32),
                     jax.ShapeDtypeStruct((_NPAD, 16), jnp.float32)]
        scratch += [pltpu.VMEM((_K, 16), jnp.float32),
                    pltpu.VMEM_SHARED((_NPAD, 16), jnp.float32),
                    pltpu.SemaphoreType.DMA]
    _HB = _CHP // _CHB

    @functools.partial(
        pl.kernel, out_type=out_type, mesh=_mesh,
        compiler_params=pltpu.CompilerParams(use_tc_tiling_on_sc=False),
        scratch_types=scratch)
    def _seg(*refs):
        if with_counts:
            (tl_hbm, tr_hbm, gidx_hbm, sidx_hbm, zrows_hbm, ones_hbm,
             z16_hbm, outl_hbm, outr_hbm, outca_hbm, outcb_hbm, gbuf, sbuf,
             r0, r1, tsp, acc, g0, g1, zsem, ones_sp, cacc, zsem2) = refs
        else:
            (tl_hbm, tr_hbm, gidx_hbm, sidx_hbm, zrows_hbm, outl_hbm,
             outr_hbm, gbuf, sbuf, r0, r1, tsp, acc, g0, g1, zsem) = refs
        cid = lax.axis_index("c")
        sid = lax.axis_index("s")
        rows = [r0, r1]
        gsem = [g0, g1]
        row0 = pl.multiple_of(sid * _RSUB, 8)
        zcopy = pltpu.async_copy(zrows_hbm, acc.at[pl.ds(row0, _RSUB)], zsem)
        if with_counts:
            zcopy2 = pltpu.async_copy(
                z16_hbm, cacc.at[pl.ds(row0, _RSUB)], zsem2)
            pltpu.sync_copy(ones_hbm, ones_sp)

        @pl.when(cid == 0)
        def _():
            pltpu.sync_copy(tl_hbm.at[pl.ds(row0, _RSUB)],
                            tsp.at[pl.ds(row0, _RSUB)])

        @pl.when(cid == 1)
        def _():
            pltpu.sync_copy(tr_hbm.at[pl.ds(row0, _RSUB)],
                            tsp.at[pl.ds(row0, _RSUB)])
        zcopy.wait()
        if with_counts:
            zcopy2.wait()
        plsc.subcore_barrier()

        def gather_start(j, b):
            pltpu.async_copy(tsp.at[gbuf.at[j]], rows[b], gsem[b])

        def gather_wait(j, b):
            pltpu.make_async_copy(tsp.at[gbuf.at[j]], rows[b], gsem[b]).wait()

        for h in range(_HB):
            pltpu.sync_copy(gidx_hbm.at[sid, pl.ds(h * _CHB, _CHB)], gbuf)
            pltpu.sync_copy(sidx_hbm.at[sid, pl.ds(h * _CHB, _CHB)], sbuf)
            gather_start(0, 0)
            gather_start(1, 1)
            count_core = 0 if h < _HB // 2 else 1

            def pair(i, carry):
                for b in range(2):
                    j = i * 2 + b
                    gather_wait(j, b)
                    pltpu.sync_copy(rows[b], acc.at[sbuf.at[j]], add=True)
                    if with_counts:
                        @pl.when(cid == count_core)
                        def _():
                            pltpu.sync_copy(ones_sp, cacc.at[sbuf.at[j]],
                                            add=True)

                    @pl.when(i < _CHB // 2 - 1)
                    def _():
                        gather_start(j + 2, b)
                return carry
            lax.fori_loop(0, _CHB // 2, pair, 0)

        plsc.subcore_barrier()

        @pl.when(cid == 0)
        def _():
            pltpu.sync_copy(acc.at[pl.ds(row0, _RSUB)],
                            outl_hbm.at[pl.ds(row0, _RSUB)])
            if with_counts:
                pltpu.sync_copy(cacc.at[pl.ds(row0, _RSUB)],
                                outca_hbm.at[pl.ds(row0, _RSUB)])

        @pl.when(cid == 1)
        def _():
            pltpu.sync_copy(acc.at[pl.ds(row0, _RSUB)],
                            outr_hbm.at[pl.ds(row0, _RSUB)])
            if with_counts:
                pltpu.sync_copy(cacc.at[pl.ds(row0, _RSUB)],
                                outcb_hbm.at[pl.ds(row0, _RSUB)])

    return _seg


_sc_seg64c = _make_sc_segsum(64, with_counts=True)
_sc_seg24 = _make_sc_segsum(24)


def _tc1_body(x_ref, w_ref, ol_ref, or_ref):
    d = jnp.dot(x_ref[...], w_ref[...], preferred_element_type=jnp.float32)
    ol_ref[...] = d[:, :64]
    or_ref[...] = d[:, 64:]


def _mean(pl_ref, pr_ref, ca_ref, cb_ref):
    cnt = (ca_ref[...] + cb_ref[...])[:, :1]
    s = jnp.concatenate([pl_ref[...], pr_ref[...]], axis=1)
    return s / jnp.maximum(cnt, 1.0)


def _tc2_body(pl_ref, pr_ref, ca_ref, cb_ref, w_ref, ol_ref, or_ref):
    yc = jnp.maximum(_mean(pl_ref, pr_ref, ca_ref, cb_ref), 0.0)
    d = jnp.dot(yc, w_ref[...], preferred_element_type=jnp.float32)
    ol_ref[...] = d[:, :64]
    or_ref[...] = d[:, 64:]


def _tc3_body(pl_ref, pr_ref, ca_ref, cb_ref, g_ref, b_ref, w_ref, ol_ref,
              or_ref):
    xv = jnp.maximum(_mean(pl_ref, pr_ref, ca_ref, cb_ref), 0.0)
    xv = xv * (g_ref[...] * _BN_SCALE) + b_ref[...]
    d = jnp.dot(xv, w_ref[...], preferred_element_type=jnp.float32)
    ol_ref[...] = d[:, :24]
    or_ref[...] = d[:, 24:48]


def _tc4_body(rl_ref, rr_ref, ca_ref, cb_ref, w_ref, ol_ref, or_ref):
    yc = jnp.maximum(_mean(rl_ref, rr_ref, ca_ref, cb_ref), 0.0)
    d = jnp.dot(yc, w_ref[...], preferred_element_type=jnp.float32)
    ol_ref[...] = d[:, :24]
    or_ref[...] = d[:, 24:48]


def _tc5_body(rl_ref, rr_ref, ca_ref, cb_ref, o_ref):
    z = _mean(rl_ref, rr_ref, ca_ref, cb_ref)
    col = lax.broadcasted_iota(jnp.int32, (_R, 48), 1)
    valid = col < 40
    zm = jnp.where(valid, z, -jnp.inf)
    m = jnp.max(zm, axis=1, keepdims=True)
    ez = jnp.where(valid, jnp.exp(z - m), 0.0)
    lse = jnp.log(jnp.sum(ez, axis=1, keepdims=True))
    o_ref[...] = jnp.where(valid, z - m - lse, 0.0)


_GRID = (_NPAD // _R,)


def _blk(c):
    return pl.BlockSpec((_R, c), lambda i: (i, 0))


def _wblk(cin, cout):
    return pl.BlockSpec((cin, cout), lambda i: (0, 0))


def _f32(*shape):
    return jax.ShapeDtypeStruct(shape, jnp.float32)


def _tc_matmul(xp, w):
    return pl.pallas_call(
        _tc1_body, grid=_GRID,
        in_specs=[_blk(128), _wblk(128, 128)],
        out_specs=[_blk(64), _blk(64)],
        out_shape=[_f32(_NPAD, 64), _f32(_NPAD, 64)],
    )(xp, w)


def _tc_mean_relu_mm(pL, pR, cA, cB, w):
    return pl.pallas_call(
        _tc2_body, grid=_GRID,
        in_specs=[_blk(64), _blk(64), _blk(16), _blk(16), _wblk(128, 128)],
        out_specs=[_blk(64), _blk(64)],
        out_shape=[_f32(_NPAD, 64), _f32(_NPAD, 64)],
    )(pL, pR, cA, cB, w)


def _tc_mean_relu_bn_mm(pL, pR, cA, cB, gam, bet, w):
    return pl.pallas_call(
        _tc3_body, grid=_GRID,
        in_specs=[_blk(64), _blk(64), _blk(16), _blk(16),
                  pl.BlockSpec((1, 128), lambda i: (0, 0)),
                  pl.BlockSpec((1, 128), lambda i: (0, 0)),
                  _wblk(128, 48)],
        out_specs=[_blk(24), _blk(24)],
        out_shape=[_f32(_NPAD, 24), _f32(_NPAD, 24)],
    )(pL, pR, cA, cB, gam, bet, w)


def _tc_mean_relu_mm48(rL, rR, cA, cB, w):
    return pl.pallas_call(
        _tc4_body, grid=_GRID,
        in_specs=[_blk(24), _blk(24), _blk(16), _blk(16), _wblk(48, 48)],
        out_specs=[_blk(24), _blk(24)],
        out_shape=[_f32(_NPAD, 24), _f32(_NPAD, 24)],
    )(rL, rR, cA, cB, w)


def _tc_mean_logsoftmax(rL, rR, cA, cB):
    return pl.pallas_call(
        _tc5_body, grid=_GRID,
        in_specs=[_blk(24), _blk(24), _blk(16), _blk(16)],
        out_specs=_blk(48),
        out_shape=_f32(_NPAD, 48),
    )(rL, rR, cA, cB)


def kernel(x, edge_index, W1_v2e, W1_e2v, bn1_gamma, bn1_beta, W2_v2e, W2_e2v):
    f32 = jnp.float32
    i32 = jnp.int32
    vidx = edge_index[0]
    eidx = edge_index[1]
    gpad = jnp.zeros((_EP - _E,), i32)
    spad = jnp.full((_EP - _E,), _TRASH, i32)
    v3 = jnp.concatenate([vidx, gpad]).reshape(_NS, _CHP, _K)
    e3 = jnp.concatenate([eidx, gpad]).reshape(_NS, _CHP, _K)
    vs3 = jnp.concatenate([vidx, spad]).reshape(_NS, _CHP, _K)
    es3 = jnp.concatenate([eidx, spad]).reshape(_NS, _CHP, _K)
    xp = jnp.zeros((_NPAD, 128), f32).at[:_N, :].set(x)
    w2v = jnp.zeros((128, 48), f32).at[:, :40].set(W2_v2e)
    w2e = jnp.zeros((48, 48), f32).at[:40, :40].set(W2_e2v)
    gam = bn1_gamma.reshape(1, 128)
    bet = bn1_beta.reshape(1, 128)
    ones16 = jnp.zeros((_K, 16), f32).at[:, 0].set(1.0)
    z16 = jnp.zeros((_RSUB, 16), f32)
    z64 = jnp.zeros((_RSUB, 64), f32)
    z24 = jnp.zeros((_RSUB, 24), f32)

    tL, tR = _tc_matmul(xp, W1_v2e)
    p1L, p1R, ceA, ceB = _sc_seg64c(tL, tR, v3, es3, z64, ones16, z16)
    yL, yR = _tc_mean_relu_mm(p1L, p1R, ceA, ceB, W1_e2v)
    p2L, p2R, cvA, cvB = _sc_seg64c(yL, yR, e3, vs3, z64, ones16, z16)
    qL, qR = _tc_mean_relu_bn_mm(p2L, p2R, cvA, cvB, gam, bet, w2v)
    r1L, r1R = _sc_seg24(qL, qR, v3, es3, z24)
    sL, sR = _tc_mean_relu_mm48(r1L, r1R, ceA, ceB, w2e)
    r2L, r2R = _sc_seg24(sL, sR, e3, vs3, z24)
    out = _tc_mean_logsoftmax(r2L, r2R, cvA, cvB)
    return out[:_N, :40]


# speedup vs baseline: 1.0954x; 1.0954x over previous
import functools

import jax
import jax.numpy as jnp
import numpy as np
from jax import lax
from jax.experimental import pallas as pl
from jax.experimental.pallas import tpu as pltpu
from jax.experimental.pallas import tpu_sc as plsc

_N = 10000
_E = 320000
_NPAD = 10240
_NC = 2
_NS = 16
_K = 128
_CHP = 160
_CHB = 40
_EP = _NS * _CHP * _K
_TRASH = 10200
_RSUB = _NPAD // _NS
_R = 2048
_BN_SCALE = float(1.0 / np.sqrt(1.0 + 1e-5))

_mesh = plsc.VectorSubcoreMesh(
    core_axis_name="c", subcore_axis_name="s", num_cores=_NC, num_subcores=_NS)


def _make_sc_segsum(W):
    @functools.partial(
        pl.kernel,
        out_type=[jax.ShapeDtypeStruct((_NPAD, W), jnp.float32),
                  jax.ShapeDtypeStruct((_NPAD, W), jnp.float32)],
        mesh=_mesh,
        compiler_params=pltpu.CompilerParams(use_tc_tiling_on_sc=False),
        scratch_types=[pltpu.VMEM((_CHB, _K), jnp.int32),
                       pltpu.VMEM((_CHB, _K), jnp.int32),
                       pltpu.VMEM((_K, W), jnp.float32),
                       pltpu.VMEM((_K, W), jnp.float32),
                       pltpu.VMEM_SHARED((_NPAD, W), jnp.float32),
                       pltpu.VMEM_SHARED((_NPAD, W), jnp.float32),
                       pltpu.SemaphoreType.DMA,
                       pltpu.SemaphoreType.DMA,
                       pltpu.SemaphoreType.DMA])
    def _seg(tl_hbm, tr_hbm, gidx_hbm, sidx_hbm, zrows_hbm, outl_hbm,
             outr_hbm, gbuf, sbuf, r0, r1, tsp, acc, g0, g1, zsem):
        cid = lax.axis_index("c")
        sid = lax.axis_index("s")
        rows = [r0, r1]
        gsem = [g0, g1]
        row0 = pl.multiple_of(sid * _RSUB, 8)
        zcopy = pltpu.async_copy(zrows_hbm, acc.at[pl.ds(row0, _RSUB)], zsem)

        @pl.when(cid == 0)
        def _():
            pltpu.sync_copy(tl_hbm.at[pl.ds(row0, _RSUB)],
                            tsp.at[pl.ds(row0, _RSUB)])

        @pl.when(cid == 1)
        def _():
            pltpu.sync_copy(tr_hbm.at[pl.ds(row0, _RSUB)],
                            tsp.at[pl.ds(row0, _RSUB)])
        zcopy.wait()
        plsc.subcore_barrier()

        def gather_start(j, b):
            pltpu.async_copy(tsp.at[gbuf.at[j]], rows[b], gsem[b])

        def gather_wait(j, b):
            pltpu.make_async_copy(tsp.at[gbuf.at[j]], rows[b], gsem[b]).wait()

        for h in range(_CHP // _CHB):
            pltpu.sync_copy(gidx_hbm.at[sid, pl.ds(h * _CHB, _CHB)], gbuf)
            pltpu.sync_copy(sidx_hbm.at[sid, pl.ds(h * _CHB, _CHB)], sbuf)
            gather_start(0, 0)
            gather_start(1, 1)

            def pair(i, carry):
                for b in range(2):
                    j = i * 2 + b
                    gather_wait(j, b)
                    pltpu.sync_copy(rows[b], acc.at[sbuf.at[j]], add=True)

                    @pl.when(i < _CHB // 2 - 1)
                    def _():
                        gather_start(j + 2, b)
                return carry
            lax.fori_loop(0, _CHB // 2, pair, 0)

        plsc.subcore_barrier()

        @pl.when(cid == 0)
        def _():
            pltpu.sync_copy(acc.at[pl.ds(row0, _RSUB)],
                            outl_hbm.at[pl.ds(row0, _RSUB)])

        @pl.when(cid == 1)
        def _():
            pltpu.sync_copy(acc.at[pl.ds(row0, _RSUB)],
                            outr_hbm.at[pl.ds(row0, _RSUB)])

    return _seg


_sc_seg64 = _make_sc_segsum(64)
_sc_seg24 = _make_sc_segsum(24)


@functools.partial(
    pl.kernel,
    out_type=[jax.ShapeDtypeStruct((_NPAD, 16), jnp.float32),
              jax.ShapeDtypeStruct((_NPAD, 16), jnp.float32)],
    mesh=_mesh,
    compiler_params=pltpu.CompilerParams(use_tc_tiling_on_sc=False),
    scratch_types=[pltpu.VMEM((_CHB, _K), jnp.int32),
                   pltpu.VMEM((_K, 16), jnp.float32),
                   pltpu.VMEM_SHARED((_NPAD, 16), jnp.float32),
                   pltpu.SemaphoreType.DMA])
def _sc_counts(eidx_hbm, vidx_hbm, ones_hbm, zrows_hbm, oute_hbm, outv_hbm,
               sbuf, ones_sp, acc, zsem):
    cid = lax.axis_index("c")
    sid = lax.axis_index("s")
    row0 = pl.multiple_of(sid * _RSUB, 8)
    zcopy = pltpu.async_copy(zrows_hbm, acc.at[pl.ds(row0, _RSUB)], zsem)
    pltpu.sync_copy(ones_hbm, ones_sp)
    zcopy.wait()
    plsc.subcore_barrier()

    for h in range(_CHP // _CHB):
        @pl.when(cid == 0)
        def _():
            pltpu.sync_copy(eidx_hbm.at[sid, pl.ds(h * _CHB, _CHB)], sbuf)

        @pl.when(cid == 1)
        def _():
            pltpu.sync_copy(vidx_hbm.at[sid, pl.ds(h * _CHB, _CHB)], sbuf)

        def chunk(j, carry):
            pltpu.sync_copy(ones_sp, acc.at[sbuf.at[j]], add=True)
            return carry
        lax.fori_loop(0, _CHB, chunk, 0)

    plsc.subcore_barrier()

    @pl.when(cid == 0)
    def _():
        pltpu.sync_copy(acc.at[pl.ds(row0, _RSUB)],
                        oute_hbm.at[pl.ds(row0, _RSUB)])

    @pl.when(cid == 1)
    def _():
        pltpu.sync_copy(acc.at[pl.ds(row0, _RSUB)],
                        outv_hbm.at[pl.ds(row0, _RSUB)])


def _tc1_body(x_ref, w_ref, ol_ref, or_ref):
    d = jnp.dot(x_ref[...], w_ref[...], preferred_element_type=jnp.float32)
    ol_ref[...] = d[:, :64]
    or_ref[...] = d[:, 64:]


def _mean(pl_ref, pr_ref, c_ref):
    cnt = c_ref[...][:, :1]
    s = jnp.concatenate([pl_ref[...], pr_ref[...]], axis=1)
    return s / jnp.maximum(cnt, 1.0)


def _tc2_body(pl_ref, pr_ref, c_ref, w_ref, ol_ref, or_ref):
    yc = jnp.maximum(_mean(pl_ref, pr_ref, c_ref), 0.0)
    d = jnp.dot(yc, w_ref[...], preferred_element_type=jnp.float32)
    ol_ref[...] = d[:, :64]
    or_ref[...] = d[:, 64:]


def _tc3_body(pl_ref, pr_ref, c_ref, g_ref, b_ref, w_ref, ol_ref, or_ref):
    xv = jnp.maximum(_mean(pl_ref, pr_ref, c_ref), 0.0)
    xv = xv * (g_ref[...] * _BN_SCALE) + b_ref[...]
    d = jnp.dot(xv, w_ref[...], preferred_element_type=jnp.float32)
    ol_ref[...] = d[:, :24]
    or_ref[...] = d[:, 24:48]


def _tc4_body(rl_ref, rr_ref, c_ref, w_ref, ol_ref, or_ref):
    yc = jnp.maximum(_mean(rl_ref, rr_ref, c_ref), 0.0)
    d = jnp.dot(yc, w_ref[...], preferred_element_type=jnp.float32)
    ol_ref[...] = d[:, :24]
    or_ref[...] = d[:, 24:48]


def _tc5_body(rl_ref, rr_ref, c_ref, o_ref):
    z = _mean(rl_ref, rr_ref, c_ref)
    col = lax.broadcasted_iota(jnp.int32, (_R, 48), 1)
    valid = col < 40
    zm = jnp.where(valid, z, -jnp.inf)
    m = jnp.max(zm, axis=1, keepdims=True)
    ez = jnp.where(valid, jnp.exp(z - m), 0.0)
    lse = jnp.log(jnp.sum(ez, axis=1, keepdims=True))
    o_ref[...] = jnp.where(valid, z - m - lse, 0.0)


_GRID = (_NPAD // _R,)


def _blk(c):
    return pl.BlockSpec((_R, c), lambda i: (i, 0))


def _wblk(cin, cout):
    return pl.BlockSpec((cin, cout), lambda i: (0, 0))


def _f32(*shape):
    return jax.ShapeDtypeStruct(shape, jnp.float32)


def _tc_matmul(xp, w):
    return pl.pallas_call(
        _tc1_body, grid=_GRID,
        in_specs=[_blk(128), _wblk(128, 128)],
        out_specs=[_blk(64), _blk(64)],
        out_shape=[_f32(_NPAD, 64), _f32(_NPAD, 64)],
    )(xp, w)


def _tc_mean_relu_mm(pL, pR, cnt, w):
    return pl.pallas_call(
        _tc2_body, grid=_GRID,
        in_specs=[_blk(64), _blk(64), _blk(16), _wblk(128, 128)],
        out_specs=[_blk(64), _blk(64)],
        out_shape=[_f32(_NPAD, 64), _f32(_NPAD, 64)],
    )(pL, pR, cnt, w)


def _tc_mean_relu_bn_mm(pL, pR, cnt, gam, bet, w):
    return pl.pallas_call(
        _tc3_body, grid=_GRID,
        in_specs=[_blk(64), _blk(64), _blk(16),
                  pl.BlockSpec((1, 128), lambda i: (0, 0)),
                  pl.BlockSpec((1, 128), lambda i: (0, 0)),
                  _wblk(128, 48)],
        out_specs=[_blk(24), _blk(24)],
        out_shape=[_f32(_NPAD, 24), _f32(_NPAD, 24)],
    )(pL, pR, cnt, gam, bet, w)


def _tc_mean_relu_mm48(rL, rR, cnt, w):
    return pl.pallas_call(
        _tc4_body, grid=_GRID,
        in_specs=[_blk(24), _blk(24), _blk(16), _wblk(48, 48)],
        out_specs=[_blk(24), _blk(24)],
        out_shape=[_f32(_NPAD, 24), _f32(_NPAD, 24)],
    )(rL, rR, cnt, w)


def _tc_mean_logsoftmax(rL, rR, cnt):
    return pl.pallas_call(
        _tc5_body, grid=_GRID,
        in_specs=[_blk(24), _blk(24), _blk(16)],
        out_specs=_blk(48),
        out_shape=_f32(_NPAD, 48),
    )(rL, rR, cnt)


def kernel(x, edge_index, W1_v2e, W1_e2v, bn1_gamma, bn1_beta, W2_v2e, W2_e2v):
    f32 = jnp.float32
    i32 = jnp.int32
    vidx = edge_index[0]
    eidx = edge_index[1]
    gpad = jnp.zeros((_EP - _E,), i32)
    spad = jnp.full((_EP - _E,), _TRASH, i32)
    v3 = jnp.concatenate([vidx, gpad]).reshape(_NS, _CHP, _K)
    e3 = jnp.concatenate([eidx, gpad]).reshape(_NS, _CHP, _K)
    vs3 = jnp.concatenate([vidx, spad]).reshape(_NS, _CHP, _K)
    es3 = jnp.concatenate([eidx, spad]).reshape(_NS, _CHP, _K)
    xp = jnp.zeros((_NPAD, 128), f32).at[:_N, :].set(x)
    w2v = jnp.zeros((128, 48), f32).at[:, :40].set(W2_v2e)
    w2e = jnp.zeros((48, 48), f32).at[:40, :40].set(W2_e2v)
    gam = bn1_gamma.reshape(1, 128)
    bet = bn1_beta.reshape(1, 128)
    ones16 = jnp.zeros((_K, 16), f32).at[:, 0].set(1.0)
    z16 = jnp.zeros((_RSUB, 16), f32)
    z64 = jnp.zeros((_RSUB, 64), f32)
    z24 = jnp.zeros((_RSUB, 24), f32)

    ce, cv = _sc_counts(es3, vs3, ones16, z16)
    tL, tR = _tc_matmul(xp, W1_v2e)
    p1L, p1R = _sc_seg64(tL, tR, v3, es3, z64)
    yL, yR = _tc_mean_relu_mm(p1L, p1R, ce, W1_e2v)
    p2L, p2R = _sc_seg64(yL, yR, e3, vs3, z64)
    qL, qR = _tc_mean_relu_bn_mm(p2L, p2R, cv, gam, bet, w2v)
    r1L, r1R = _sc_seg24(qL, qR, v3, es3, z24)
    sL, sR = _tc_mean_relu_mm48(r1L, r1R, ce, w2e)
    r2L, r2R = _sc_seg24(sL, sR, e3, vs3, z24)
    out = _tc_mean_logsoftmax(r2L, r2R, cv)
    return out[:_N, :40]


# speedup vs baseline: 1.1276x; 1.0294x over previous
import functools

import jax
import jax.numpy as jnp
import numpy as np
from jax import lax
from jax.experimental import pallas as pl
from jax.experimental.pallas import tpu as pltpu
from jax.experimental.pallas import tpu_sc as plsc

_N = 10000
_E = 320000
_NPAD = 10240
_NC = 2
_NS = 16
_K = 128
_CHP = 160
_CHB = 80
_EP = _NS * _CHP * _K
_TRASH = 10200
_RSUB = _NPAD // _NS
_R = 2048
_BN_SCALE = float(1.0 / np.sqrt(1.0 + 1e-5))

_mesh = plsc.VectorSubcoreMesh(
    core_axis_name="c", subcore_axis_name="s", num_cores=_NC, num_subcores=_NS)


def _make_sc_segsum(W):
    @functools.partial(
        pl.kernel,
        out_type=[jax.ShapeDtypeStruct((_NPAD, W), jnp.float32),
                  jax.ShapeDtypeStruct((_NPAD, W), jnp.float32)],
        mesh=_mesh,
        compiler_params=pltpu.CompilerParams(use_tc_tiling_on_sc=False),
        scratch_types=[pltpu.VMEM((_CHB, _K), jnp.int32),
                       pltpu.VMEM((_CHB, _K), jnp.int32),
                       pltpu.VMEM((_K, W), jnp.float32),
                       pltpu.VMEM((_K, W), jnp.float32),
                       pltpu.VMEM_SHARED((_NPAD, W), jnp.float32),
                       pltpu.VMEM_SHARED((_NPAD, W), jnp.float32),
                       pltpu.SemaphoreType.DMA,
                       pltpu.SemaphoreType.DMA,
                       pltpu.SemaphoreType.DMA])
    def _seg(tl_hbm, tr_hbm, gidx_hbm, sidx_hbm, zrows_hbm, outl_hbm,
             outr_hbm, gbuf, sbuf, r0, r1, tsp, acc, g0, g1, zsem):
        cid = lax.axis_index("c")
        sid = lax.axis_index("s")
        rows = [r0, r1]
        gsem = [g0, g1]
        row0 = pl.multiple_of(sid * _RSUB, 8)
        zcopy = pltpu.async_copy(zrows_hbm, acc.at[pl.ds(row0, _RSUB)], zsem)

        @pl.when(cid == 0)
        def _():
            pltpu.sync_copy(tl_hbm.at[pl.ds(row0, _RSUB)],
                            tsp.at[pl.ds(row0, _RSUB)])

        @pl.when(cid == 1)
        def _():
            pltpu.sync_copy(tr_hbm.at[pl.ds(row0, _RSUB)],
                            tsp.at[pl.ds(row0, _RSUB)])
        zcopy.wait()
        plsc.subcore_barrier()

        def gather_start(j, b):
            pltpu.async_copy(tsp.at[gbuf.at[j]], rows[b], gsem[b])

        def gather_wait(j, b):
            pltpu.make_async_copy(tsp.at[gbuf.at[j]], rows[b], gsem[b]).wait()

        for h in range(_CHP // _CHB):
            pltpu.sync_copy(gidx_hbm.at[sid, pl.ds(h * _CHB, _CHB)], gbuf)
            pltpu.sync_copy(sidx_hbm.at[sid, pl.ds(h * _CHB, _CHB)], sbuf)
            gather_start(0, 0)
            gather_start(1, 1)

            def pair(i, carry):
                for b in range(2):
                    j = i * 2 + b
                    gather_wait(j, b)
                    pltpu.sync_copy(rows[b], acc.at[sbuf.at[j]], add=True)

                    @pl.when(i < _CHB // 2 - 1)
                    def _():
                        gather_start(j + 2, b)
                return carry
            lax.fori_loop(0, _CHB // 2, pair, 0)

        plsc.subcore_barrier()

        @pl.when(cid == 0)
        def _():
            pltpu.sync_copy(acc.at[pl.ds(row0, _RSUB)],
                            outl_hbm.at[pl.ds(row0, _RSUB)])

        @pl.when(cid == 1)
        def _():
            pltpu.sync_copy(acc.at[pl.ds(row0, _RSUB)],
                            outr_hbm.at[pl.ds(row0, _RSUB)])

    return _seg


_sc_seg64 = _make_sc_segsum(64)
_sc_seg24 = _make_sc_segsum(24)


@functools.partial(
    pl.kernel,
    out_type=[jax.ShapeDtypeStruct((_NPAD, 16), jnp.float32),
              jax.ShapeDtypeStruct((_NPAD, 16), jnp.float32)],
    mesh=_mesh,
    compiler_params=pltpu.CompilerParams(use_tc_tiling_on_sc=False),
    scratch_types=[pltpu.VMEM((_CHB, _K), jnp.int32),
                   pltpu.VMEM((_K, 16), jnp.float32),
                   pltpu.VMEM_SHARED((_NPAD, 16), jnp.float32),
                   pltpu.SemaphoreType.DMA])
def _sc_counts(eidx_hbm, vidx_hbm, ones_hbm, zrows_hbm, oute_hbm, outv_hbm,
               sbuf, ones_sp, acc, zsem):
    cid = lax.axis_index("c")
    sid = lax.axis_index("s")
    row0 = pl.multiple_of(sid * _RSUB, 8)
    zcopy = pltpu.async_copy(zrows_hbm, acc.at[pl.ds(row0, _RSUB)], zsem)
    pltpu.sync_copy(ones_hbm, ones_sp)
    zcopy.wait()
    plsc.subcore_barrier()

    for h in range(_CHP // _CHB):
        @pl.when(cid == 0)
        def _():
            pltpu.sync_copy(eidx_hbm.at[sid, pl.ds(h * _CHB, _CHB)], sbuf)

        @pl.when(cid == 1)
        def _():
            pltpu.sync_copy(vidx_hbm.at[sid, pl.ds(h * _CHB, _CHB)], sbuf)

        def chunk(j, carry):
            pltpu.sync_copy(ones_sp, acc.at[sbuf.at[j]], add=True)
            return carry
        lax.fori_loop(0, _CHB, chunk, 0)

    plsc.subcore_barrier()

    @pl.when(cid == 0)
    def _():
        pltpu.sync_copy(acc.at[pl.ds(row0, _RSUB)],
                        oute_hbm.at[pl.ds(row0, _RSUB)])

    @pl.when(cid == 1)
    def _():
        pltpu.sync_copy(acc.at[pl.ds(row0, _RSUB)],
                        outv_hbm.at[pl.ds(row0, _RSUB)])


def _tc1_body(x_ref, w_ref, ol_ref, or_ref):
    d = jnp.dot(x_ref[...], w_ref[...], preferred_element_type=jnp.float32)
    ol_ref[...] = d[:, :64]
    or_ref[...] = d[:, 64:]


def _mean(pl_ref, pr_ref, c_ref):
    cnt = c_ref[...][:, :1]
    s = jnp.concatenate([pl_ref[...], pr_ref[...]], axis=1)
    return s / jnp.maximum(cnt, 1.0)


def _tc2_body(pl_ref, pr_ref, c_ref, w_ref, ol_ref, or_ref):
    yc = jnp.maximum(_mean(pl_ref, pr_ref, c_ref), 0.0)
    d = jnp.dot(yc, w_ref[...], preferred_element_type=jnp.float32)
    ol_ref[...] = d[:, :64]
    or_ref[...] = d[:, 64:]


def _tc3_body(pl_ref, pr_ref, c_ref, g_ref, b_ref, w_ref, ol_ref, or_ref):
    xv = jnp.maximum(_mean(pl_ref, pr_ref, c_ref), 0.0)
    xv = xv * (g_ref[...] * _BN_SCALE) + b_ref[...]
    d = jnp.dot(xv, w_ref[...], preferred_element_type=jnp.float32)
    ol_ref[...] = d[:, :24]
    or_ref[...] = d[:, 24:48]


def _tc4_body(rl_ref, rr_ref, c_ref, w_ref, ol_ref, or_ref):
    yc = jnp.maximum(_mean(rl_ref, rr_ref, c_ref), 0.0)
    d = jnp.dot(yc, w_ref[...], preferred_element_type=jnp.float32)
    ol_ref[...] = d[:, :24]
    or_ref[...] = d[:, 24:48]


def _tc5_body(rl_ref, rr_ref, c_ref, o_ref):
    z = _mean(rl_ref, rr_ref, c_ref)
    col = lax.broadcasted_iota(jnp.int32, (_R, 48), 1)
    valid = col < 40
    zm = jnp.where(valid, z, -jnp.inf)
    m = jnp.max(zm, axis=1, keepdims=True)
    ez = jnp.where(valid, jnp.exp(z - m), 0.0)
    lse = jnp.log(jnp.sum(ez, axis=1, keepdims=True))
    o_ref[...] = jnp.where(valid, z - m - lse, 0.0)


_GRID = (_NPAD // _R,)


def _blk(c):
    return pl.BlockSpec((_R, c), lambda i: (i, 0))


def _wblk(cin, cout):
    return pl.BlockSpec((cin, cout), lambda i: (0, 0))


def _f32(*shape):
    return jax.ShapeDtypeStruct(shape, jnp.float32)


def _tc_matmul(xp, w):
    return pl.pallas_call(
        _tc1_body, grid=_GRID,
        in_specs=[_blk(128), _wblk(128, 128)],
        out_specs=[_blk(64), _blk(64)],
        out_shape=[_f32(_NPAD, 64), _f32(_NPAD, 64)],
    )(xp, w)


def _tc_mean_relu_mm(pL, pR, cnt, w):
    return pl.pallas_call(
        _tc2_body, grid=_GRID,
        in_specs=[_blk(64), _blk(64), _blk(16), _wblk(128, 128)],
        out_specs=[_blk(64), _blk(64)],
        out_shape=[_f32(_NPAD, 64), _f32(_NPAD, 64)],
    )(pL, pR, cnt, w)


def _tc_mean_relu_bn_mm(pL, pR, cnt, gam, bet, w):
    return pl.pallas_call(
        _tc3_body, grid=_GRID,
        in_specs=[_blk(64), _blk(64), _blk(16),
                  pl.BlockSpec((1, 128), lambda i: (0, 0)),
                  pl.BlockSpec((1, 128), lambda i: (0, 0)),
                  _wblk(128, 48)],
        out_specs=[_blk(24), _blk(24)],
        out_shape=[_f32(_NPAD, 24), _f32(_NPAD, 24)],
    )(pL, pR, cnt, gam, bet, w)


def _tc_mean_relu_mm48(rL, rR, cnt, w):
    return pl.pallas_call(
        _tc4_body, grid=_GRID,
        in_specs=[_blk(24), _blk(24), _blk(16), _wblk(48, 48)],
        out_specs=[_blk(24), _blk(24)],
        out_shape=[_f32(_NPAD, 24), _f32(_NPAD, 24)],
    )(rL, rR, cnt, w)


def _tc_mean_logsoftmax(rL, rR, cnt):
    return pl.pallas_call(
        _tc5_body, grid=_GRID,
        in_specs=[_blk(24), _blk(24), _blk(16)],
        out_specs=_blk(48),
        out_shape=_f32(_NPAD, 48),
    )(rL, rR, cnt)


def kernel(x, edge_index, W1_v2e, W1_e2v, bn1_gamma, bn1_beta, W2_v2e, W2_e2v):
    f32 = jnp.float32
    i32 = jnp.int32
    vidx = edge_index[0]
    eidx = edge_index[1]
    gpad = jnp.zeros((_EP - _E,), i32)
    spad = jnp.full((_EP - _E,), _TRASH, i32)
    v3 = jnp.concatenate([vidx, gpad]).reshape(_NS, _CHP, _K)
    e3 = jnp.concatenate([eidx, gpad]).reshape(_NS, _CHP, _K)
    vs3 = jnp.concatenate([vidx, spad]).reshape(_NS, _CHP, _K)
    es3 = jnp.concatenate([eidx, spad]).reshape(_NS, _CHP, _K)
    xp = jnp.zeros((_NPAD, 128), f32).at[:_N, :].set(x)
    w2v = jnp.zeros((128, 48), f32).at[:, :40].set(W2_v2e)
    w2e = jnp.zeros((48, 48), f32).at[:40, :40].set(W2_e2v)
    gam = bn1_gamma.reshape(1, 128)
    bet = bn1_beta.reshape(1, 128)
    ones16 = jnp.zeros((_K, 16), f32).at[:, 0].set(1.0)
    z16 = jnp.zeros((_RSUB, 16), f32)
    z64 = jnp.zeros((_RSUB, 64), f32)
    z24 = jnp.zeros((_RSUB, 24), f32)

    ce, cv = _sc_counts(es3, vs3, ones16, z16)
    tL, tR = _tc_matmul(xp, W1_v2e)
    p1L, p1R = _sc_seg64(tL, tR, v3, es3, z64)
    yL, yR = _tc_mean_relu_mm(p1L, p1R, ce, W1_e2v)
    p2L, p2R = _sc_seg64(yL, yR, e3, vs3, z64)
    qL, qR = _tc_mean_relu_bn_mm(p2L, p2R, cv, gam, bet, w2v)
    r1L, r1R = _sc_seg24(qL, qR, v3, es3, z24)
    sL, sR = _tc_mean_relu_mm48(r1L, r1R, ce, w2e)
    r2L, r2R = _sc_seg24(sL, sR, e3, vs3, z24)
    out = _tc_mean_logsoftmax(r2L, r2R, cv)
    return out[:_N, :40]
